# baseline (speedup 1.0000x reference)
import functools

import jax
import jax.numpy as jnp
from jax import lax
from jax.experimental import pallas as pl
from jax.experimental.pallas import tpu as pltpu
from jax.experimental.pallas import tpu_sc as plsc

VS_ = 1000000
DS_ = 64
B_ = 16384
C_ = 20

NC = 2
NS = 16
NW = NC * NS
CHUNK = 128
NBUF = 8

WRD_CH_W = (B_ // CHUNK) // NW
CTX_CH_W = (B_ * C_ // CHUNK) // NW


def _sc_gather_body(wrd_idx, ctx_idx, wi, wo, out_w, out_c, widx, cidx,
                    rows, *sems):
    gsems = sems[:NBUF]
    osems = sems[NBUF:]
    wid = lax.axis_index("s") * NC + lax.axis_index("c")

    pltpu.sync_copy(wrd_idx.at[pl.ds(wid * WRD_CH_W, WRD_CH_W)], widx)
    pltpu.sync_copy(ctx_idx.at[pl.ds(wid * CTX_CH_W, CTX_CH_W)], cidx)

    def stream(idx_v, table, out, base_chunk, n_chunks):
        nb = min(NBUF, n_chunks)

        def start_gather(j, b):
            pltpu.async_copy(table.at[idx_v.at[j]], rows.at[b], gsems[b])

        def wait_gather(b):
            pltpu.make_async_copy(table.at[idx_v.at[0]], rows.at[b],
                                  gsems[b]).wait()

        def start_out(j, b):
            pltpu.async_copy(
                rows.at[b], out.at[pl.ds((base_chunk + j) * CHUNK, CHUNK)],
                osems[b])

        def wait_out(b):
            pltpu.make_async_copy(rows.at[b], out.at[pl.ds(0, CHUNK)],
                                  osems[b]).wait()

        for b in range(nb):
            start_gather(b, b)

        n_groups = n_chunks // nb
        if n_groups > 1:
            def group(gi, carry):
                for b in range(nb):
                    j = gi * nb + b
                    wait_gather(b)
                    start_out(j, b)
                    wait_out(b)
                    start_gather(j + nb, b)
                return carry
            lax.fori_loop(0, n_groups - 1, group, 0)

        for b in range(nb):
            j = (n_groups - 1) * nb + b
            wait_gather(b)
            start_out(j, b)
        for b in range(nb):
            wait_out(b)

    stream(widx, wi, out_w, wid * WRD_CH_W, WRD_CH_W)
    stream(cidx, wo, out_c, wid * CTX_CH_W, CTX_CH_W)


RB = 512


def _tc_loss_body(wrd_ref, ctx_ref, pos_ref, neg_ref, out_ref):
    w = wrd_ref[...]
    cx = ctx_ref[...]
    p = pos_ref[...]
    n = neg_ref[...]
    e = jnp.sum(w[:, None, :] * cx, axis=2)
    e = e * (1.0 - 2.0 * n)
    sg = 1.0 / (1.0 + jnp.exp(-e))
    l = -jnp.log(jnp.clip(sg, 1e-6, 1.0 - 1e-6))
    pe = jnp.sum(l * p, axis=1) / jnp.sum(p, axis=1)
    ne = jnp.sum(l * n, axis=1)
    blk = jnp.sum(pe) + jnp.sum(ne)

    @pl.when(pl.program_id(0) == 0)
    def _():
        out_ref[0, 0] = 0.0

    out_ref[0, 0] += blk


@functools.lru_cache(maxsize=1)
def _build_sc_gather():
    mesh = plsc.VectorSubcoreMesh(core_axis_name="c", subcore_axis_name="s")
    return pl.kernel(
        _sc_gather_body,
        out_type=(jax.ShapeDtypeStruct((B_, DS_), jnp.float32),
                  jax.ShapeDtypeStruct((B_ * C_, DS_), jnp.float32)),
        mesh=mesh,
        scratch_types=[
            pltpu.VMEM((WRD_CH_W, CHUNK), jnp.int32),
            pltpu.VMEM((CTX_CH_W, CHUNK), jnp.int32),
            pltpu.VMEM((NBUF, CHUNK, DS_), jnp.float32),
        ] + [pltpu.SemaphoreType.DMA] * (2 * NBUF),
        compiler_params=pltpu.CompilerParams(use_tc_tiling_on_sc=False),
    )


@functools.lru_cache(maxsize=1)
def _build_tc_loss():
    grid = B_ // RB
    return pl.pallas_call(
        _tc_loss_body,
        grid=(grid,),
        in_specs=[
            pl.BlockSpec((RB, DS_), lambda i: (i, 0)),
            pl.BlockSpec((RB, C_, DS_), lambda i: (i, 0, 0)),
            pl.BlockSpec((RB, C_), lambda i: (i, 0)),
            pl.BlockSpec((RB, C_), lambda i: (i, 0)),
        ],
        out_specs=pl.BlockSpec((1, 1), lambda i: (0, 0),
                               memory_space=pltpu.SMEM),
        out_shape=jax.ShapeDtypeStruct((1, 1), jnp.float32),
    )


def kernel(wrd, ctx, pos, neg, W_i, W_o):
    wrd2d = wrd.astype(jnp.int32).reshape(B_ // CHUNK, CHUNK)
    ctx2d = ctx.astype(jnp.int32).reshape(B_ * C_ // CHUNK, CHUNK)
    wrd_emb, ctx_emb = _build_sc_gather()(wrd2d, ctx2d, W_i, W_o)
    tot = _build_tc_loss()(wrd_emb, ctx_emb.reshape(B_, C_, DS_), pos, neg)
    return tot[0, 0] / B_


# speedup vs baseline: 1.3528x; 1.3528x over previous
import functools

import jax
import jax.numpy as jnp
from jax import lax
from jax.experimental import pallas as pl
from jax.experimental.pallas import tpu as pltpu
from jax.experimental.pallas import tpu_sc as plsc

VS_ = 1000000
DS_ = 64
B_ = 16384
C_ = 20

NC = 2
NS = 16
NW = NC * NS
CHUNK = 128
NBUF = 8

B_W = B_ // NW
WRD_CH_W = B_W // CHUNK
CTX_CH_W = (B_ * C_ // CHUNK) // NW
ERR_W = B_W * C_


def _sc_gather_body(wrd_idx, ctx_idx, wi, wo, out_e, widx, cidx,
                    wrows, rows, estage, *sems):
    gsems = sems[:NBUF]
    wid = lax.axis_index("s") * NC + lax.axis_index("c")

    pltpu.sync_copy(wrd_idx.at[pl.ds(wid * WRD_CH_W, WRD_CH_W)], widx)
    pltpu.sync_copy(ctx_idx.at[pl.ds(wid * CTX_CH_W, CTX_CH_W)], cidx)

    for c in range(WRD_CH_W):
        pltpu.async_copy(wi.at[widx.at[c]], wrows.at[pl.ds(c * CHUNK, CHUNK)],
                         gsems[0])
    for c in range(WRD_CH_W):
        pltpu.make_async_copy(wi.at[widx.at[0]],
                              wrows.at[pl.ds(0, CHUNK)], gsems[0]).wait()

    def start_gather(j, b):
        pltpu.async_copy(wo.at[cidx.at[j]], rows.at[b], gsems[b])

    def wait_gather(b):
        pltpu.make_async_copy(wo.at[cidx.at[0]], rows.at[b], gsems[b]).wait()

    lane = lax.iota(jnp.int32, 16)
    last = jnp.full((16,), 15, jnp.int32)

    def consume(j, b):
        def grp(g, carry):
            r0 = g * 16
            vec = jnp.zeros((16,), jnp.float32)
            for t in range(16):
                r = r0 + t
                bl = (j * CHUNK + r) // C_
                acc = (wrows[bl, pl.ds(0, 16)] * rows[b, r, pl.ds(0, 16)]
                       + wrows[bl, pl.ds(16, 16)] * rows[b, r, pl.ds(16, 16)]
                       + wrows[bl, pl.ds(32, 16)] * rows[b, r, pl.ds(32, 16)]
                       + wrows[bl, pl.ds(48, 16)] * rows[b, r, pl.ds(48, 16)])
                cs = plsc.cumsum(acc)
                sv = cs.at[last].get(mode="promise_in_bounds")
                vec = jnp.where(lane == t, sv, vec)
            estage[pl.ds(j * CHUNK + r0, 16)] = vec
            return carry
        lax.fori_loop(0, CHUNK // 16, grp, 0)

    for b in range(NBUF):
        start_gather(b, b)

    n_groups = CTX_CH_W // NBUF

    def group(gi, carry):
        for b in range(NBUF):
            j = gi * NBUF + b
            wait_gather(b)
            consume(j, b)
            start_gather(j + NBUF, b)
        return carry
    lax.fori_loop(0, n_groups - 1, group, 0)

    for b in range(NBUF):
        j = (n_groups - 1) * NBUF + b
        wait_gather(b)
        consume(j, b)

    pltpu.sync_copy(estage, out_e.at[pl.ds(wid * ERR_W, ERR_W)])


RB = 2048


def _tc_loss_body(err_ref, pos_ref, neg_ref, out_ref):
    e = err_ref[...]
    p = pos_ref[...]
    n = neg_ref[...]
    e = e * (1.0 - 2.0 * n)
    sg = 1.0 / (1.0 + jnp.exp(-e))
    l = -jnp.log(jnp.clip(sg, 1e-6, 1.0 - 1e-6))
    pe = jnp.sum(l * p, axis=1) / jnp.sum(p, axis=1)
    ne = jnp.sum(l * n, axis=1)
    blk = jnp.sum(pe) + jnp.sum(ne)

    @pl.when(pl.program_id(0) == 0)
    def _():
        out_ref[0, 0] = 0.0

    out_ref[0, 0] += blk


@functools.lru_cache(maxsize=1)
def _build_sc_gather():
    mesh = plsc.VectorSubcoreMesh(core_axis_name="c", subcore_axis_name="s")
    return pl.kernel(
        _sc_gather_body,
        out_type=jax.ShapeDtypeStruct((B_ * C_,), jnp.float32),
        mesh=mesh,
        scratch_types=[
            pltpu.VMEM((WRD_CH_W, CHUNK), jnp.int32),
            pltpu.VMEM((CTX_CH_W, CHUNK), jnp.int32),
            pltpu.VMEM((B_W, DS_), jnp.float32),
            pltpu.VMEM((NBUF, CHUNK, DS_), jnp.float32),
            pltpu.VMEM((ERR_W,), jnp.float32),
        ] + [pltpu.SemaphoreType.DMA] * NBUF,
        compiler_params=pltpu.CompilerParams(use_tc_tiling_on_sc=False,
                                             needs_layout_passes=False),
    )


@functools.lru_cache(maxsize=1)
def _build_tc_loss():
    grid = B_ // RB
    return pl.pallas_call(
        _tc_loss_body,
        grid=(grid,),
        in_specs=[
            pl.BlockSpec((RB, C_), lambda i: (i, 0)),
            pl.BlockSpec((RB, C_), lambda i: (i, 0)),
            pl.BlockSpec((RB, C_), lambda i: (i, 0)),
        ],
        out_specs=pl.BlockSpec((1, 1), lambda i: (0, 0),
                               memory_space=pltpu.SMEM),
        out_shape=jax.ShapeDtypeStruct((1, 1), jnp.float32),
    )


def kernel(wrd, ctx, pos, neg, W_i, W_o):
    wrd2d = wrd.astype(jnp.int32).reshape(B_ // CHUNK, CHUNK)
    ctx2d = ctx.astype(jnp.int32).reshape(B_ * C_ // CHUNK, CHUNK)
    err = _build_sc_gather()(wrd2d, ctx2d, W_i, W_o)
    tot = _build_tc_loss()(err.reshape(B_, C_), pos, neg)
    return tot[0, 0] / B_
